# baseline (speedup 1.0000x reference)
import jax
import jax.numpy as jnp
from jax.experimental import pallas as pl
from jax.experimental.pallas import tpu as pltpu

_N = 2048
_D = 1024
_F = 2048
_TF = 512


def _ffn_block(x_ref, w1_ref, b1_ref, w2_ref, b2_ref, o_ref, xs_ref):
    f = pl.program_id(0)

    @pl.when(f == 0)
    def _cache_x():
        xs_ref[...] = x_ref[...].astype(jnp.bfloat16)

    h = jnp.dot(xs_ref[...], w1_ref[0].astype(jnp.bfloat16),
                preferred_element_type=jnp.float32)
    h = h + b1_ref[0]
    h = 0.5 * h * (1.0 + jax.lax.erf(h * 0.7071067811865476))
    p = jnp.dot(h.astype(jnp.bfloat16), w2_ref[0].astype(jnp.bfloat16),
                preferred_element_type=jnp.float32)

    @pl.when(f == 0)
    def _init_out():
        o_ref[...] = p + b2_ref[0]

    @pl.when(f != 0)
    def _accum_out():
        o_ref[...] = o_ref[...] + p


def _ffn(xf, w1, b1, w2, b2):
    grid = (_F // _TF,)
    return pl.pallas_call(
        _ffn_block,
        grid=grid,
        in_specs=[
            pl.BlockSpec((_N, _D), lambda f: (0, 0)),
            pl.BlockSpec((1, _D, _TF), lambda f: (0, 0, f)),
            pl.BlockSpec((1, 1, _TF), lambda f: (0, 0, f)),
            pl.BlockSpec((1, _TF, _D), lambda f: (0, f, 0)),
            pl.BlockSpec((1, 1, _D), lambda f: (0, 0, 0)),
        ],
        out_specs=pl.BlockSpec((_N, _D), lambda f: (0, 0)),
        out_shape=jax.ShapeDtypeStruct((_N, _D), jnp.float32),
        scratch_shapes=[
            pltpu.VMEM((_N, _D), jnp.bfloat16),
        ],
    )(xf, w1, b1, w2, b2)


def kernel(x, edge_index, W_gat, att_src, att_dst, bias_gat, ln_gamma, ln_beta,
           W1, b1, W2, b2):
    B, N, D = x.shape
    xf = x.reshape(B * N, D)
    out = _ffn(xf, W1, b1.reshape(b1.shape[0], 1, -1), W2,
               b2.reshape(b2.shape[0], 1, -1))
    return out.reshape(B, N, D)


# speedup vs baseline: 1.0106x; 1.0106x over previous
import jax
import jax.numpy as jnp
from jax.experimental import pallas as pl
from jax.experimental.pallas import tpu as pltpu

_N = 2048
_D = 1024
_F = 2048
_TM = 256
_NT = _N // _TM


def _ffn_block(x_ref, w1_ref, b1_ref, w2_ref, b2_ref, o_ref,
               h_s, w1_s, w2_s):
    i = pl.program_id(0)

    @pl.when(i == 0)
    def _cache_w1():
        w1_s[...] = w1_ref[0].astype(jnp.bfloat16)

    @pl.when(i < _NT)
    def _phase_a():
        w2_s[pl.ds(i * _TM, _TM), :] = w2_ref[0].astype(jnp.bfloat16)
        h = jnp.dot(x_ref[...].astype(jnp.bfloat16), w1_s[...],
                    preferred_element_type=jnp.float32)
        h = h + b1_ref[0]
        h = 0.5 * h * (1.0 + jax.lax.erf(h * 0.7071067811865476))
        h_s[pl.ds(i * _TM, _TM), :] = h.astype(jnp.bfloat16)

    @pl.when(i >= _NT)
    def _phase_b():
        m = i - _NT
        o = jnp.dot(h_s[pl.ds(m * _TM, _TM), :], w2_s[...],
                    preferred_element_type=jnp.float32)
        o_ref[...] = o + b2_ref[0]


def _ffn(xf, w1, b1, w2, b2):
    grid = (2 * _NT,)
    return pl.pallas_call(
        _ffn_block,
        grid=grid,
        in_specs=[
            pl.BlockSpec((_TM, _D), lambda i: (jnp.minimum(i, _NT - 1), 0)),
            pl.BlockSpec((1, _D, _F), lambda i: (0, 0, 0)),
            pl.BlockSpec((1, 1, _F), lambda i: (0, 0, 0)),
            pl.BlockSpec((1, _TM, _D),
                         lambda i: (0, jnp.minimum(i, _NT - 1), 0)),
            pl.BlockSpec((1, 1, _D), lambda i: (0, 0, 0)),
        ],
        out_specs=pl.BlockSpec(
            (_TM, _D), lambda i: (jnp.maximum(i, _NT) - _NT, 0)),
        out_shape=jax.ShapeDtypeStruct((_N, _D), jnp.float32),
        scratch_shapes=[
            pltpu.VMEM((_N, _F), jnp.bfloat16),
            pltpu.VMEM((_D, _F), jnp.bfloat16),
            pltpu.VMEM((_F, _D), jnp.bfloat16),
        ],
    )(xf, w1, b1, w2, b2)


def kernel(x, edge_index, W_gat, att_src, att_dst, bias_gat, ln_gamma, ln_beta,
           W1, b1, W2, b2):
    B, N, D = x.shape
    xf = x.reshape(B * N, D)
    out = _ffn(xf, W1, b1.reshape(b1.shape[0], 1, -1), W2,
               b2.reshape(b2.shape[0], 1, -1))
    return out.reshape(B, N, D)
